# baseline (speedup 1.0000x reference)
import functools

import jax
import jax.numpy as jnp
from jax import lax
from jax.experimental import pallas as pl
from jax.experimental.pallas import tpu as pltpu
from jax.experimental.pallas import tpu_sc as plsc

_NC = 2
_NS = 16
_NW = _NC * _NS
_CHUNK = 128


def _sc_lookup(idx2d, table):
    n_rows, chunk = idx2d.shape
    B = n_rows * chunk
    D = table.shape[1]
    rows_per_w = n_rows // _NW
    b_per_w = rows_per_w * chunk

    mesh = plsc.VectorSubcoreMesh(core_axis_name="c", subcore_axis_name="s")

    @functools.partial(
        pl.kernel,
        mesh=mesh,
        out_type=jax.ShapeDtypeStruct((B, D), table.dtype),
        scratch_types=[
            pltpu.VMEM((rows_per_w, chunk), jnp.int32),
            pltpu.VMEM((b_per_w, D), jnp.float32),
            pltpu.SemaphoreType.DMA,
        ],
    )
    def k(idx_hbm, table_hbm, out_hbm, idx_v, rows_v, sem):
        wid = lax.axis_index("s") * _NC + lax.axis_index("c")
        pltpu.sync_copy(idx_hbm.at[pl.ds(wid * rows_per_w, rows_per_w)], idx_v)
        copies = [
            pltpu.async_copy(
                table_hbm.at[idx_v.at[j]],
                rows_v.at[pl.ds(j * chunk, chunk)],
                sem,
            )
            for j in range(rows_per_w)
        ]
        for c in copies:
            c.wait()
        pltpu.sync_copy(rows_v, out_hbm.at[pl.ds(wid * b_per_w, b_per_w)])

    return k(idx2d, table)


def kernel(inputs, table):
    B = inputs.shape[0]
    idx2d = inputs.astype(jnp.int32).reshape(B // _CHUNK, _CHUNK)
    return _sc_lookup(idx2d, table)


# speedup vs baseline: 1.3173x; 1.3173x over previous
import functools

import jax
import jax.numpy as jnp
from jax import lax
from jax.experimental import pallas as pl
from jax.experimental.pallas import tpu as pltpu
from jax.experimental.pallas import tpu_sc as plsc

_NC = 2
_NS = 16
_NW = _NC * _NS
_L = 16


def _sc_lookup(idx, ttab, B, D):
    b_per_w = B // _NW
    n_grp = b_per_w // _L

    mesh = plsc.VectorSubcoreMesh(core_axis_name="c", subcore_axis_name="s")

    @functools.partial(
        pl.kernel,
        mesh=mesh,
        compiler_params=pltpu.CompilerParams(needs_layout_passes=False),
        out_type=jax.ShapeDtypeStruct((B * D,), jnp.float32),
        scratch_types=[
            pltpu.VMEM((b_per_w,), jnp.int32),
            pltpu.VMEM((D, _L), jnp.float32),
            pltpu.VMEM((b_per_w * D,), jnp.float32),
            pltpu.SemaphoreType.DMA,
        ],
    )
    def k(idx_hbm, ttab_hbm, out_hbm, idx_v, ttab_v, buf, sem):
        wid = lax.axis_index("s") * _NC + lax.axis_index("c")
        base = wid * b_per_w
        pltpu.sync_copy(idx_hbm.at[pl.ds(base, b_per_w)], idx_v)
        pltpu.sync_copy(ttab_hbm, ttab_v)
        lane_d = lax.iota(jnp.int32, _L) * D

        def body(i, carry):
            idxv = idx_v[pl.ds(i * _L, _L)]
            bvec = lane_d + i * (_L * D)
            for c in range(D):
                tcol = ttab_v[c]
                vals = tcol.at[idxv].get(mode="promise_in_bounds")
                plsc.store_scatter(buf, [bvec + c], vals)
            return carry

        lax.fori_loop(0, n_grp, body, 0)
        pltpu.sync_copy(buf, out_hbm.at[pl.ds(base * D, b_per_w * D)])

    return k(idx, ttab)


def kernel(inputs, table):
    B = inputs.shape[0]
    V, D = table.shape
    idx = inputs.astype(jnp.int32).reshape(B)
    ttab = jnp.pad(table, ((0, _L - V), (0, 0))).T
    out = _sc_lookup(idx, ttab, B, D)
    return out.reshape(B, D)


# speedup vs baseline: 1.7064x; 1.2954x over previous
import functools

import jax
import jax.numpy as jnp
from jax import lax
from jax.experimental import pallas as pl
from jax.experimental.pallas import tpu as pltpu
from jax.experimental.pallas import tpu_sc as plsc

_NC = 2
_NS = 16
_NW = _NC * _NS
_L = 16


def _sc_lookup(idx, ttab, B, D):
    b_per_w = B // _NW
    n_grp = b_per_w // _L

    mesh = plsc.VectorSubcoreMesh(core_axis_name="c", subcore_axis_name="s")

    @functools.partial(
        pl.kernel,
        mesh=mesh,
        compiler_params=pltpu.CompilerParams(needs_layout_passes=False),
        out_type=jax.ShapeDtypeStruct((B * D,), jnp.float32),
        scratch_types=[
            pltpu.VMEM((b_per_w,), jnp.int32),
            pltpu.VMEM((D, _L), jnp.float32),
            pltpu.VMEM((b_per_w * D,), jnp.float32),
            pltpu.SemaphoreType.DMA,
        ],
    )
    def k(idx_hbm, ttab_hbm, out_hbm, idx_v, ttab_v, buf, sem):
        wid = lax.axis_index("s") * _NC + lax.axis_index("c")
        base = wid * b_per_w
        pltpu.sync_copy(idx_hbm.at[pl.ds(base, b_per_w)], idx_v)
        pltpu.sync_copy(ttab_hbm, ttab_v)
        lane_d = lax.iota(jnp.int32, _L) * D

        def body(i, carry):
            idxv = idx_v[pl.ds(i * _L, _L)]
            bvec = lane_d + i * (_L * D)

            @plsc.parallel_loop(0, D, unroll=16)
            def col_loop(c):
                tcol = ttab_v[c]
                vals = tcol.at[idxv].get(mode="promise_in_bounds")
                plsc.store_scatter(buf, [bvec + c], vals)

            return carry

        lax.fori_loop(0, n_grp, body, 0)
        pltpu.sync_copy(buf, out_hbm.at[pl.ds(base * D, b_per_w * D)])

    return k(idx, ttab)


def kernel(inputs, table):
    B = inputs.shape[0]
    V, D = table.shape
    idx = inputs.astype(jnp.int32).reshape(B)
    ttab = jnp.pad(table, ((0, _L - V), (0, 0))).T
    out = _sc_lookup(idx, ttab, B, D)
    return out.reshape(B, D)


# speedup vs baseline: 3.1872x; 1.8678x over previous
import functools

import jax
import jax.numpy as jnp
from jax import lax
from jax.experimental import pallas as pl
from jax.experimental.pallas import tpu as pltpu
from jax.experimental.pallas import tpu_sc as plsc

_NC = 2
_NS = 16
_NW = _NC * _NS
_L = 16


def _sc_lookup(idx, table, B, V, D):
    b_per_w = B // _NW
    n_vec = D // _L

    mesh = plsc.VectorSubcoreMesh(core_axis_name="c", subcore_axis_name="s")

    @functools.partial(
        pl.kernel,
        mesh=mesh,
        compiler_params=pltpu.CompilerParams(needs_layout_passes=False),
        out_type=jax.ShapeDtypeStruct((B * D,), jnp.float32),
        scratch_types=[
            pltpu.VMEM((b_per_w,), jnp.int32),
            pltpu.VMEM((V * D,), jnp.float32),
            pltpu.VMEM((b_per_w * D,), jnp.float32),
            pltpu.SemaphoreType.DMA,
        ],
    )
    def k(idx_hbm, tab_hbm, out_hbm, idx_v, tab_v, buf, sem):
        wid = lax.axis_index("s") * _NC + lax.axis_index("c")
        base = wid * b_per_w
        pltpu.sync_copy(idx_hbm.at[pl.ds(base, b_per_w)], idx_v)
        pltpu.sync_copy(tab_hbm, tab_v)

        @plsc.parallel_loop(0, b_per_w // _L, unroll=1)
        def grp_loop(g):
            idxv = idx_v[pl.ds(g * _L, _L)]
            for l in range(_L):
                row = idxv[l] * D
                out_off = (g * _L + l) * D
                for q in range(n_vec):
                    buf[pl.ds(out_off + q * _L, _L)] = (
                        tab_v[pl.ds(row + q * _L, _L)])

        pltpu.sync_copy(buf, out_hbm.at[pl.ds(base * D, b_per_w * D)])

    return k(idx, table)


def kernel(inputs, table):
    B = inputs.shape[0]
    V, D = table.shape
    idx = inputs.astype(jnp.int32).reshape(B)
    out = _sc_lookup(idx, table.reshape(V * D), B, V, D)
    return out.reshape(B, D)


# speedup vs baseline: 3.2928x; 1.0331x over previous
import functools

import jax
import jax.numpy as jnp
from jax import lax
from jax.experimental import pallas as pl
from jax.experimental.pallas import tpu as pltpu
from jax.experimental.pallas import tpu_sc as plsc

_NC = 2
_NS = 16
_NW = _NC * _NS
_L = 16


def _sc_lookup(idx, table, B, V, D):
    b_per_w = B // _NW
    n_vec = D // _L

    mesh = plsc.VectorSubcoreMesh(core_axis_name="c", subcore_axis_name="s")

    @functools.partial(
        pl.kernel,
        mesh=mesh,
        compiler_params=pltpu.CompilerParams(needs_layout_passes=False),
        out_type=jax.ShapeDtypeStruct((B * D,), jnp.float32),
        scratch_types=[
            pltpu.VMEM((b_per_w,), jnp.int32),
            pltpu.VMEM((V * D,), jnp.float32),
            pltpu.VMEM((b_per_w * D,), jnp.float32),
            pltpu.SemaphoreType.DMA,
        ],
    )
    def k(idx_hbm, tab_hbm, out_hbm, idx_v, tab_v, buf, sem):
        wid = lax.axis_index("s") * _NC + lax.axis_index("c")
        base = wid * b_per_w
        pltpu.sync_copy(idx_hbm.at[pl.ds(base, b_per_w)], idx_v)
        pltpu.sync_copy(tab_hbm, tab_v)

        n_chunk = 4
        g_per_chunk = (b_per_w // _L) // n_chunk
        w_per_chunk = (b_per_w // n_chunk) * D

        def chunk_body(ck, carry):
            @plsc.parallel_loop(0, g_per_chunk, unroll=1)
            def grp_loop(g):
                gg = ck * g_per_chunk + g
                idxv = idx_v[pl.ds(gg * _L, _L)]
                for l in range(_L):
                    row = idxv[l] * D
                    out_off = (gg * _L + l) * D
                    for q in range(n_vec):
                        buf[pl.ds(out_off + q * _L, _L)] = (
                            tab_v[pl.ds(row + q * _L, _L)])

            pltpu.async_copy(
                buf.at[pl.ds(ck * w_per_chunk, w_per_chunk)],
                out_hbm.at[pl.ds(base * D + ck * w_per_chunk, w_per_chunk)],
                sem,
            )
            return carry

        lax.fori_loop(0, n_chunk, chunk_body, 0)
        for ck in range(n_chunk):
            pltpu.make_async_copy(
                buf.at[pl.ds(ck * w_per_chunk, w_per_chunk)],
                out_hbm.at[pl.ds(base * D + ck * w_per_chunk, w_per_chunk)],
                sem,
            ).wait()

    return k(idx, table)


def kernel(inputs, table):
    B = inputs.shape[0]
    V, D = table.shape
    idx = inputs.astype(jnp.int32).reshape(B)
    out = _sc_lookup(idx, table.reshape(V * D), B, V, D)
    return out.reshape(B, D)


# speedup vs baseline: 3.3025x; 1.0030x over previous
import functools

import jax
import jax.numpy as jnp
from jax import lax
from jax.experimental import pallas as pl
from jax.experimental.pallas import tpu as pltpu
from jax.experimental.pallas import tpu_sc as plsc

_NC = 2
_NS = 16
_NW = _NC * _NS
_L = 16


def _sc_lookup(idx, table, B, V, D):
    b_per_w = B // _NW
    n_vec = D // _L

    mesh = plsc.VectorSubcoreMesh(core_axis_name="c", subcore_axis_name="s")

    @functools.partial(
        pl.kernel,
        mesh=mesh,
        compiler_params=pltpu.CompilerParams(needs_layout_passes=False),
        out_type=jax.ShapeDtypeStruct((B * D,), jnp.float32),
        scratch_types=[
            pltpu.VMEM((b_per_w,), jnp.int32),
            pltpu.VMEM((V * D,), jnp.float32),
            pltpu.VMEM((b_per_w * D,), jnp.float32),
            pltpu.SemaphoreType.DMA,
        ],
    )
    def k(idx_hbm, tab_hbm, out_hbm, idx_v, tab_v, buf, sem):
        wid = lax.axis_index("s") * _NC + lax.axis_index("c")
        base = wid * b_per_w
        pltpu.sync_copy(idx_hbm.at[pl.ds(base, b_per_w)], idx_v)
        pltpu.sync_copy(tab_hbm, tab_v)

        n_chunk = 4
        g_per_chunk = (b_per_w // _L) // n_chunk
        w_per_chunk = (b_per_w // n_chunk) * D

        def chunk_body(ck, carry):
            @plsc.parallel_loop(0, g_per_chunk, unroll=1)
            def grp_loop(g):
                gg = ck * g_per_chunk + g
                idxv = idx_v[pl.ds(gg * _L, _L)]
                rows = [idxv[l] * D for l in range(_L)]
                for l in range(_L):
                    out_off = (gg * _L + l) * D
                    for q in range(n_vec):
                        buf[pl.ds(out_off + q * _L, _L)] = (
                            tab_v[pl.ds(rows[l] + q * _L, _L)])

            pltpu.async_copy(
                buf.at[pl.ds(ck * w_per_chunk, w_per_chunk)],
                out_hbm.at[pl.ds(base * D + ck * w_per_chunk, w_per_chunk)],
                sem,
            )
            return carry

        lax.fori_loop(0, n_chunk, chunk_body, 0)
        for ck in range(n_chunk):
            pltpu.make_async_copy(
                buf.at[pl.ds(ck * w_per_chunk, w_per_chunk)],
                out_hbm.at[pl.ds(base * D + ck * w_per_chunk, w_per_chunk)],
                sem,
            ).wait()

    return k(idx, table)


def kernel(inputs, table):
    B = inputs.shape[0]
    V, D = table.shape
    idx = inputs.astype(jnp.int32).reshape(B)
    out = _sc_lookup(idx, table.reshape(V * D), B, V, D)
    return out.reshape(B, D)
